# baseline (speedup 1.0000x reference)
import jax
import jax.numpy as jnp
from jax import lax
from jax.experimental import pallas as pl
from jax.experimental.pallas import tpu as pltpu
from jax.experimental.pallas import tpu_sc as plsc

_N = 10000
_E = 320000
_DIN = 128
_H = 64

_NC = 2
_NS = 16
_NW = _NC * _NS
_C = 256
_EPAD = 327680
_PW = _EPAD // _NW
_NCH = _PW // _C
_NACC = 10240
_RPS = _NACC // _NS
_ZR = 128
_DEGW = 16


def _dotb(a, b):
    return lax.dot_general(a.astype(jnp.bfloat16), b.astype(jnp.bfloat16),
                           (((1,), (0,)), ((), ())),
                           preferred_element_type=jnp.float32)


def _make_sc_agg(nsrc, with_deg, staged=False, nbuf=4):
    mesh = plsc.VectorSubcoreMesh(core_axis_name="c", subcore_axis_name="s")
    out_types = [jax.ShapeDtypeStruct((_NC, _N, _H), jnp.float32)
                 for _ in range(nsrc)]
    scratch = [
        pltpu.VMEM((_NCH, _C), jnp.int32),
        pltpu.VMEM((_NCH, _C), jnp.int32),
        pltpu.VMEM((_ZR, _H), jnp.float32),
    ]
    for _ in range(nsrc):
        scratch += [[pltpu.VMEM((_C, _H), jnp.float32) for _ in range(nbuf)],
                    pltpu.VMEM_SHARED((_NACC, _H), jnp.float32),
                    [pltpu.SemaphoreType.DMA for _ in range(nbuf)]]
        if staged:
            scratch.append(pltpu.VMEM_SHARED((_N, _H), jnp.float32))
    if with_deg:
        out_types.append(jax.ShapeDtypeStruct((_NC, _N, _DEGW), jnp.float32))
        scratch += [
            pltpu.VMEM((_ZR, _DEGW), jnp.float32),
            pltpu.VMEM((_C, _DEGW), jnp.float32),
            pltpu.VMEM_SHARED((_NACC, _DEGW), jnp.float32),
        ]

    def body(*refs):
        y_hbm = refs[:nsrc]
        col_hbm, row_hbm = refs[nsrc:nsrc + 2]
        refs = refs[nsrc + 2:]
        acc_hbm = refs[:nsrc]
        refs = refs[nsrc:]
        if with_deg:
            deg_hbm = refs[0]
            refs = refs[1:]
        colv, rowv, zbuf = refs[:3]
        refs = refs[3:]
        stride = 4 if staged else 3
        bufs, acc_sh, sems, src_sh = [], [], [], []
        for s in range(nsrc):
            bufs.append(refs[stride * s])
            acc_sh.append(refs[stride * s + 1])
            sems.append(refs[stride * s + 2])
            if staged:
                src_sh.append(refs[stride * s + 3])
        refs = refs[stride * nsrc:]
        if with_deg:
            dzbuf, onesb, deg_sh = refs

        cid = lax.axis_index("c")
        sid = lax.axis_index("s")
        wid = cid * _NS + sid

        z16 = jnp.zeros((16,), jnp.float32)

        @pl.loop(0, _ZR)
        def _(r):
            @pl.loop(0, _H, step=16)
            def _(c0):
                zbuf[r, pl.ds(c0, 16)] = z16

        for s in range(nsrc):
            @pl.loop(0, _RPS, step=_ZR)
            def _(r0, s=s):
                pltpu.sync_copy(zbuf, acc_sh[s].at[pl.ds(sid * _RPS + r0, _ZR)])

        if with_deg:
            one16 = jnp.ones((16,), jnp.float32)

            @pl.loop(0, _ZR)
            def _(r):
                dzbuf[r, pl.ds(0, _DEGW)] = z16

            @pl.loop(0, _C)
            def _(r):
                onesb[r, pl.ds(0, _DEGW)] = one16

            @pl.loop(0, _RPS, step=_ZR)
            def _(r0):
                pltpu.sync_copy(dzbuf, deg_sh.at[pl.ds(sid * _RPS + r0, _ZR)])

        if staged:
            rps = _N // _NS
            for s in range(nsrc):
                pltpu.sync_copy(y_hbm[s].at[pl.ds(sid * rps, rps)],
                                src_sh[s].at[pl.ds(sid * rps, rps)])
            src = src_sh
        else:
            src = y_hbm

        plsc.subcore_barrier()

        pltpu.sync_copy(col_hbm.at[pl.ds(wid * _NCH, _NCH)], colv)
        pltpu.sync_copy(row_hbm.at[pl.ds(wid * _NCH, _NCH)], rowv)

        for k in range(nbuf):
            for s in range(nsrc):
                pltpu.async_copy(src[s].at[colv.at[k]], bufs[s][k],
                                 sems[s][k])

        @pl.loop(0, _NCH // nbuf)
        def _(j):
            for k in range(nbuf):
                i = nbuf * j + k
                for s in range(nsrc):
                    pltpu.make_async_copy(src[s].at[colv.at[i]], bufs[s][k],
                                          sems[s][k]).wait()
                    pltpu.sync_copy(bufs[s][k], acc_sh[s].at[rowv.at[i]],
                                    add=True)
                if with_deg:
                    pltpu.sync_copy(onesb, deg_sh.at[rowv.at[i]], add=True)

                @pl.when(i + nbuf < _NCH)
                def _(i=i, k=k):
                    for s in range(nsrc):
                        pltpu.async_copy(src[s].at[colv.at[i + nbuf]],
                                         bufs[s][k], sems[s][k])

        plsc.subcore_barrier()
        wps = _N // _NS
        for s in range(nsrc):
            pltpu.sync_copy(acc_sh[s].at[pl.ds(sid * wps, wps)],
                            acc_hbm[s].at[cid, pl.ds(sid * wps, wps)])
        if with_deg:
            pltpu.sync_copy(deg_sh.at[pl.ds(sid * wps, wps)],
                            deg_hbm.at[cid, pl.ds(sid * wps, wps)])

    return pl.kernel(body, out_type=out_types, mesh=mesh, scratch_types=scratch,
                     compiler_params=pltpu.CompilerParams(use_tc_tiling_on_sc=False))


_sc_agg_deg = _make_sc_agg(1, True, nbuf=2)
_sc_agg = _make_sc_agg(1, False, nbuf=2)


def _tc_layer_body(accp_ref, degm_ref, h_ref, wl_ref, wr_ref, b_ref, o_ref):
    aggn = (accp_ref[0] + accp_ref[1]) / degm_ref[...]
    o_ref[...] = jnp.maximum(
        _dotb(aggn, wl_ref[...]) + b_ref[...] + _dotb(h_ref[...], wr_ref[...]),
        0.0)


def _tc_layer0_body(accpa_ref, accpb_ref, degp_ref, x_ref, wl_ref, wr_ref,
                    b_ref, o_ref, degm_ref):
    degm = jnp.maximum(degp_ref[0, :, 0:1] + degp_ref[1, :, 0:1], 1.0)
    degm_ref[...] = degm
    agg = jnp.concatenate(
        [accpa_ref[0] + accpa_ref[1], accpb_ref[0] + accpb_ref[1]], axis=1)
    aggn = agg / degm
    o_ref[...] = jnp.maximum(
        _dotb(aggn, wl_ref[...]) + b_ref[...] + _dotb(x_ref[...], wr_ref[...]),
        0.0)


def _tc_final_body(accp_ref, degm_ref, h_ref, wlp_ref, wrdv_ref, brdv_ref,
                   o_ref):
    aggn = (accp_ref[0] + accp_ref[1]) / degm_ref[...]
    h = h_ref[...]
    hd = _dotb(h, wrdv_ref[...]) + brdv_ref[...]
    probs = _dotb(aggn, wlp_ref[...]) + hd[:, 0:1]
    o_ref[...] = jnp.concatenate([probs, hd[:, 1:3]], axis=1)


def _f32(*shape):
    return jax.ShapeDtypeStruct(shape, jnp.float32)


def kernel(x, edge_index, Wl0, bl0, Wr0, Wl1, bl1, Wr1, Wl2, bl2, Wr2,
           Wl3, bl3, Wr3, Wlp, blp, Wrp, Wdn, bdn, Wv, bv):
    row = edge_index[0]
    col = edge_index[1]
    pad = _EPAD - _E
    ar = jnp.arange(pad, dtype=jnp.int32)
    rowp = jnp.concatenate([row, _N + ar % (_NACC - _N)])
    colp = jnp.concatenate([col, ar % _N])
    rowp = rowp.reshape(_EPAD // _C, _C)
    colp = colp.reshape(_EPAD // _C, _C)

    xa = jnp.asarray(x[:, :_H])
    xb = jnp.asarray(x[:, _H:])
    accpa, degp = _sc_agg_deg(xa, colp, rowp)
    accpb = _sc_agg(xb, colp, rowp)[0]
    h, degm = pl.pallas_call(
        _tc_layer0_body, out_shape=[_f32(_N, _H), _f32(_N, 1)])(
        accpa, accpb, degp, x, Wl0, Wr0, bl0.reshape(1, _H))

    for Wl, bl, Wr in ((Wl1, bl1, Wr1), (Wl2, bl2, Wr2), (Wl3, bl3, Wr3)):
        accp = _sc_agg(h, colp, rowp)[0]
        h = pl.pallas_call(_tc_layer_body, out_shape=_f32(_N, _H))(
            accp, degm, h, Wl, Wr, bl.reshape(1, _H))

    accp = _sc_agg(h, colp, rowp)[0]
    wrdv = jnp.concatenate([Wrp, Wdn, Wv], axis=1)
    brdv = jnp.concatenate([blp, bdn, bv]).reshape(1, 3)
    return pl.pallas_call(_tc_final_body, out_shape=_f32(_N, 3))(
        accp, degm, h, Wlp, wrdv, brdv)


# speedup vs baseline: 1.0540x; 1.0540x over previous
import jax
import jax.numpy as jnp
from jax import lax
from jax.experimental import pallas as pl
from jax.experimental.pallas import tpu as pltpu
from jax.experimental.pallas import tpu_sc as plsc

_N = 10000
_E = 320000
_DIN = 128
_H = 64

_NC = 2
_NS = 16
_NW = _NC * _NS
_C = 128
_EPAD = 327680
_PW = _EPAD // _NW
_NCH = _PW // _C
_NACC = 10240
_RPS = _NACC // _NS
_ZR = 128
_DEGW = 16


def _dotb(a, b):
    return lax.dot_general(a.astype(jnp.bfloat16), b.astype(jnp.bfloat16),
                           (((1,), (0,)), ((), ())),
                           preferred_element_type=jnp.float32)


def _make_sc_agg(nsrc, with_deg, staged=False, nbuf=4):
    mesh = plsc.VectorSubcoreMesh(core_axis_name="c", subcore_axis_name="s")
    out_types = [jax.ShapeDtypeStruct((_NC, _N, _H), jnp.float32)
                 for _ in range(nsrc)]
    scratch = [
        pltpu.VMEM((_NCH, _C), jnp.int32),
        pltpu.VMEM((_NCH, _C), jnp.int32),
        pltpu.VMEM((_ZR, _H), jnp.float32),
    ]
    for _ in range(nsrc):
        scratch += [[pltpu.VMEM((_C, _H), jnp.float32) for _ in range(nbuf)],
                    pltpu.VMEM_SHARED((_NACC, _H), jnp.float32),
                    [pltpu.SemaphoreType.DMA for _ in range(nbuf)]]
        if staged:
            scratch.append(pltpu.VMEM_SHARED((_N, _H), jnp.float32))
    if with_deg:
        out_types.append(jax.ShapeDtypeStruct((_NC, _N, _DEGW), jnp.float32))
        scratch += [
            pltpu.VMEM((_ZR, _DEGW), jnp.float32),
            pltpu.VMEM((_C, _DEGW), jnp.float32),
            pltpu.VMEM_SHARED((_NACC, _DEGW), jnp.float32),
        ]

    def body(*refs):
        y_hbm = refs[:nsrc]
        col_hbm, row_hbm = refs[nsrc:nsrc + 2]
        refs = refs[nsrc + 2:]
        acc_hbm = refs[:nsrc]
        refs = refs[nsrc:]
        if with_deg:
            deg_hbm = refs[0]
            refs = refs[1:]
        colv, rowv, zbuf = refs[:3]
        refs = refs[3:]
        stride = 4 if staged else 3
        bufs, acc_sh, sems, src_sh = [], [], [], []
        for s in range(nsrc):
            bufs.append(refs[stride * s])
            acc_sh.append(refs[stride * s + 1])
            sems.append(refs[stride * s + 2])
            if staged:
                src_sh.append(refs[stride * s + 3])
        refs = refs[stride * nsrc:]
        if with_deg:
            dzbuf, onesb, deg_sh = refs

        cid = lax.axis_index("c")
        sid = lax.axis_index("s")
        wid = cid * _NS + sid

        z16 = jnp.zeros((16,), jnp.float32)

        @pl.loop(0, _ZR)
        def _(r):
            @pl.loop(0, _H, step=16)
            def _(c0):
                zbuf[r, pl.ds(c0, 16)] = z16

        for s in range(nsrc):
            @pl.loop(0, _RPS, step=_ZR)
            def _(r0, s=s):
                pltpu.sync_copy(zbuf, acc_sh[s].at[pl.ds(sid * _RPS + r0, _ZR)])

        if with_deg:
            one16 = jnp.ones((16,), jnp.float32)

            @pl.loop(0, _ZR)
            def _(r):
                dzbuf[r, pl.ds(0, _DEGW)] = z16

            @pl.loop(0, _C)
            def _(r):
                onesb[r, pl.ds(0, _DEGW)] = one16

            @pl.loop(0, _RPS, step=_ZR)
            def _(r0):
                pltpu.sync_copy(dzbuf, deg_sh.at[pl.ds(sid * _RPS + r0, _ZR)])

        if staged:
            rps = _N // _NS
            for s in range(nsrc):
                pltpu.sync_copy(y_hbm[s].at[pl.ds(sid * rps, rps)],
                                src_sh[s].at[pl.ds(sid * rps, rps)])
            src = src_sh
        else:
            src = y_hbm

        plsc.subcore_barrier()

        pltpu.sync_copy(col_hbm.at[pl.ds(wid * _NCH, _NCH)], colv)
        pltpu.sync_copy(row_hbm.at[pl.ds(wid * _NCH, _NCH)], rowv)

        for k in range(nbuf):
            for s in range(nsrc):
                pltpu.async_copy(src[s].at[colv.at[k]], bufs[s][k],
                                 sems[s][k])

        @pl.loop(0, _NCH // nbuf)
        def _(j):
            for k in range(nbuf):
                i = nbuf * j + k
                for s in range(nsrc):
                    pltpu.make_async_copy(src[s].at[colv.at[i]], bufs[s][k],
                                          sems[s][k]).wait()
                    pltpu.sync_copy(bufs[s][k], acc_sh[s].at[rowv.at[i]],
                                    add=True)
                if with_deg:
                    pltpu.sync_copy(onesb, deg_sh.at[rowv.at[i]], add=True)

                @pl.when(i + nbuf < _NCH)
                def _(i=i, k=k):
                    for s in range(nsrc):
                        pltpu.async_copy(src[s].at[colv.at[i + nbuf]],
                                         bufs[s][k], sems[s][k])

        plsc.subcore_barrier()
        wps = _N // _NS
        for s in range(nsrc):
            pltpu.sync_copy(acc_sh[s].at[pl.ds(sid * wps, wps)],
                            acc_hbm[s].at[cid, pl.ds(sid * wps, wps)])
        if with_deg:
            pltpu.sync_copy(deg_sh.at[pl.ds(sid * wps, wps)],
                            deg_hbm.at[cid, pl.ds(sid * wps, wps)])

    return pl.kernel(body, out_type=out_types, mesh=mesh, scratch_types=scratch,
                     compiler_params=pltpu.CompilerParams(use_tc_tiling_on_sc=False))


_sc_agg_deg = _make_sc_agg(1, True)
_sc_agg = _make_sc_agg(1, False)


def _tc_r_body(h_ref, wr_ref, b_ref, r_ref):
    r_ref[...] = _dotb(h_ref[...], wr_ref[...]) + b_ref[...]


def _tc_layer_body(accp_ref, degm_ref, r_ref, wl_ref, o_ref):
    aggn = (accp_ref[0] + accp_ref[1]) / degm_ref[...]
    o_ref[...] = jnp.maximum(_dotb(aggn, wl_ref[...]) + r_ref[...], 0.0)


def _tc_layer0_body(accpa_ref, accpb_ref, degp_ref, r_ref, wl_ref,
                    o_ref, degm_ref):
    degm = jnp.maximum(degp_ref[0, :, 0:1] + degp_ref[1, :, 0:1], 1.0)
    degm_ref[...] = degm
    agg = jnp.concatenate(
        [accpa_ref[0] + accpa_ref[1], accpb_ref[0] + accpb_ref[1]], axis=1)
    aggn = agg / degm
    o_ref[...] = jnp.maximum(_dotb(aggn, wl_ref[...]) + r_ref[...], 0.0)


def _tc_final_body(accp_ref, degm_ref, hd_ref, wlp_ref, o_ref):
    aggn = (accp_ref[0] + accp_ref[1]) / degm_ref[...]
    hd = hd_ref[...]
    probs = _dotb(aggn, wlp_ref[...]) + hd[:, 0:1]
    o_ref[...] = jnp.concatenate([probs, hd[:, 1:3]], axis=1)


def _f32(*shape):
    return jax.ShapeDtypeStruct(shape, jnp.float32)


def kernel(x, edge_index, Wl0, bl0, Wr0, Wl1, bl1, Wr1, Wl2, bl2, Wr2,
           Wl3, bl3, Wr3, Wlp, blp, Wrp, Wdn, bdn, Wv, bv):
    row = edge_index[0]
    col = edge_index[1]
    pad = _EPAD - _E
    ar = jnp.arange(pad, dtype=jnp.int32)
    rowp = jnp.concatenate([row, _N + ar % (_NACC - _N)])
    colp = jnp.concatenate([col, ar % _N])
    rowp = rowp.reshape(_EPAD // _C, _C)
    colp = colp.reshape(_EPAD // _C, _C)

    xa = jnp.asarray(x[:, :_H])
    xb = jnp.asarray(x[:, _H:])
    r = pl.pallas_call(_tc_r_body, out_shape=_f32(_N, _H))(
        x, Wr0, bl0.reshape(1, _H))
    accpa, degp = _sc_agg_deg(xa, colp, rowp)
    accpb = _sc_agg(xb, colp, rowp)[0]
    h, degm = pl.pallas_call(
        _tc_layer0_body, out_shape=[_f32(_N, _H), _f32(_N, 1)])(
        accpa, accpb, degp, r, Wl0)

    for Wl, bl, Wr in ((Wl1, bl1, Wr1), (Wl2, bl2, Wr2), (Wl3, bl3, Wr3)):
        r = pl.pallas_call(_tc_r_body, out_shape=_f32(_N, _H))(
            h, Wr, bl.reshape(1, _H))
        accp = _sc_agg(h, colp, rowp)[0]
        h = pl.pallas_call(_tc_layer_body, out_shape=_f32(_N, _H))(
            accp, degm, r, Wl)

    wrdv = jnp.concatenate([Wrp, Wdn, Wv], axis=1)
    brdv = jnp.concatenate([blp, bdn, bv]).reshape(1, 3)
    hd = pl.pallas_call(_tc_r_body, out_shape=_f32(_N, 3))(h, wrdv, brdv)
    accp = _sc_agg(h, colp, rowp)[0]
    return pl.pallas_call(_tc_final_body, out_shape=_f32(_N, 3))(
        accp, degm, hd, Wlp)


# speedup vs baseline: 1.0657x; 1.0110x over previous
import jax
import jax.numpy as jnp
from jax import lax
from jax.experimental import pallas as pl
from jax.experimental.pallas import tpu as pltpu
from jax.experimental.pallas import tpu_sc as plsc

_N = 10000
_E = 320000
_DIN = 128
_H = 64

_NC = 2
_NS = 16
_NW = _NC * _NS
_C = 128
_EPAD = 327680
_PW = _EPAD // _NW
_NCH = _PW // _C
_NACC = 10240
_RPS = _NACC // _NS
_ZR = 128
_DEGW = 16


def _dotb(a, b):
    return lax.dot_general(a.astype(jnp.bfloat16), b.astype(jnp.bfloat16),
                           (((1,), (0,)), ((), ())),
                           preferred_element_type=jnp.float32)


def _make_sc_agg(nsrc, with_deg, staged=False, nbuf=4):
    mesh = plsc.VectorSubcoreMesh(core_axis_name="c", subcore_axis_name="s")
    out_types = [jax.ShapeDtypeStruct((_NC, _N, _H), jnp.float32)
                 for _ in range(nsrc)]
    scratch = [
        pltpu.VMEM((_NCH, _C), jnp.int32),
        pltpu.VMEM((_NCH, _C), jnp.int32),
        pltpu.VMEM((_ZR, _H), jnp.float32),
    ]
    for _ in range(nsrc):
        scratch += [[pltpu.VMEM((_C, _H), jnp.float32) for _ in range(nbuf)],
                    pltpu.VMEM_SHARED((_NACC, _H), jnp.float32),
                    [pltpu.SemaphoreType.DMA for _ in range(nbuf)]]
        if staged:
            scratch.append(pltpu.VMEM_SHARED((_N, _H), jnp.float32))
    if with_deg:
        out_types.append(jax.ShapeDtypeStruct((_NC, _N, _DEGW), jnp.float32))
        scratch += [
            pltpu.VMEM((_ZR, _DEGW), jnp.float32),
            pltpu.VMEM((_C, _DEGW), jnp.float32),
            pltpu.VMEM_SHARED((_NACC, _DEGW), jnp.float32),
        ]

    def body(*refs):
        y_hbm = refs[:nsrc]
        col_hbm, row_hbm = refs[nsrc:nsrc + 2]
        refs = refs[nsrc + 2:]
        acc_hbm = refs[:nsrc]
        refs = refs[nsrc:]
        if with_deg:
            deg_hbm = refs[0]
            refs = refs[1:]
        colv, rowv, zbuf = refs[:3]
        refs = refs[3:]
        stride = 4 if staged else 3
        bufs, acc_sh, sems, src_sh = [], [], [], []
        for s in range(nsrc):
            bufs.append(refs[stride * s])
            acc_sh.append(refs[stride * s + 1])
            sems.append(refs[stride * s + 2])
            if staged:
                src_sh.append(refs[stride * s + 3])
        refs = refs[stride * nsrc:]
        if with_deg:
            dzbuf, onesb, deg_sh = refs

        cid = lax.axis_index("c")
        sid = lax.axis_index("s")
        wid = cid * _NS + sid

        z16 = jnp.zeros((16,), jnp.float32)

        @pl.loop(0, _ZR)
        def _(r):
            @pl.loop(0, _H, step=16)
            def _(c0):
                zbuf[r, pl.ds(c0, 16)] = z16

        for s in range(nsrc):
            @pl.loop(0, _RPS, step=_ZR)
            def _(r0, s=s):
                pltpu.sync_copy(zbuf, acc_sh[s].at[pl.ds(sid * _RPS + r0, _ZR)])

        if with_deg:
            one16 = jnp.ones((16,), jnp.float32)

            @pl.loop(0, _ZR)
            def _(r):
                dzbuf[r, pl.ds(0, _DEGW)] = z16

            @pl.loop(0, _C)
            def _(r):
                onesb[r, pl.ds(0, _DEGW)] = one16

            @pl.loop(0, _RPS, step=_ZR)
            def _(r0):
                pltpu.sync_copy(dzbuf, deg_sh.at[pl.ds(sid * _RPS + r0, _ZR)])

        if staged:
            rps = _N // _NS
            for s in range(nsrc):
                pltpu.sync_copy(y_hbm[s].at[pl.ds(sid * rps, rps)],
                                src_sh[s].at[pl.ds(sid * rps, rps)])
            src = src_sh
        else:
            src = y_hbm

        plsc.subcore_barrier()

        pltpu.sync_copy(col_hbm.at[pl.ds(wid * _NCH, _NCH)], colv)
        pltpu.sync_copy(row_hbm.at[pl.ds(wid * _NCH, _NCH)], rowv)

        for k in range(nbuf):
            for s in range(nsrc):
                pltpu.async_copy(src[s].at[colv.at[k]], bufs[s][k],
                                 sems[s][k])

        @pl.loop(0, _NCH // nbuf)
        def _(j):
            for k in range(nbuf):
                i = nbuf * j + k
                for s in range(nsrc):
                    pltpu.make_async_copy(src[s].at[colv.at[i]], bufs[s][k],
                                          sems[s][k]).wait()
                    pltpu.sync_copy(bufs[s][k], acc_sh[s].at[rowv.at[i]],
                                    add=True)
                if with_deg:
                    pltpu.sync_copy(onesb, deg_sh.at[rowv.at[i]], add=True)

                @pl.when(i + nbuf < _NCH)
                def _(i=i, k=k):
                    for s in range(nsrc):
                        pltpu.async_copy(src[s].at[colv.at[i + nbuf]],
                                         bufs[s][k], sems[s][k])

        plsc.subcore_barrier()
        wps = _N // _NS
        for s in range(nsrc):
            pltpu.sync_copy(acc_sh[s].at[pl.ds(sid * wps, wps)],
                            acc_hbm[s].at[cid, pl.ds(sid * wps, wps)])
        if with_deg:
            pltpu.sync_copy(deg_sh.at[pl.ds(sid * wps, wps)],
                            deg_hbm.at[cid, pl.ds(sid * wps, wps)])

    return pl.kernel(body, out_type=out_types, mesh=mesh, scratch_types=scratch,
                     compiler_params=pltpu.CompilerParams(use_tc_tiling_on_sc=False))


_sc_agg_deg = _make_sc_agg(1, True)
_sc_agg = _make_sc_agg(1, False)


def _tc_layer_body(accp_ref, degm_ref, h_ref, wl_ref, wr_ref, b_ref, o_ref):
    aggn = (accp_ref[0] + accp_ref[1]) / degm_ref[...]
    o_ref[...] = jnp.maximum(
        _dotb(aggn, wl_ref[...]) + b_ref[...] + _dotb(h_ref[...], wr_ref[...]),
        0.0)


def _tc_layer0_body(accpa_ref, accpb_ref, degp_ref, x_ref, wl_ref, wr_ref,
                    b_ref, o_ref, degm_ref):
    degm = jnp.maximum(degp_ref[0, :, 0:1] + degp_ref[1, :, 0:1], 1.0)
    degm_ref[...] = degm
    agg = jnp.concatenate(
        [accpa_ref[0] + accpa_ref[1], accpb_ref[0] + accpb_ref[1]], axis=1)
    aggn = agg / degm
    o_ref[...] = jnp.maximum(
        _dotb(aggn, wl_ref[...]) + b_ref[...] + _dotb(x_ref[...], wr_ref[...]),
        0.0)


def _tc_final_body(accp_ref, degm_ref, h_ref, wlp_ref, wrdv_ref, brdv_ref,
                   o_ref):
    aggn = (accp_ref[0] + accp_ref[1]) / degm_ref[...]
    h = h_ref[...]
    hd = _dotb(h, wrdv_ref[...]) + brdv_ref[...]
    probs = _dotb(aggn, wlp_ref[...]) + hd[:, 0:1]
    o_ref[...] = jnp.concatenate([probs, hd[:, 1:3]], axis=1)


def _f32(*shape):
    return jax.ShapeDtypeStruct(shape, jnp.float32)


def kernel(x, edge_index, Wl0, bl0, Wr0, Wl1, bl1, Wr1, Wl2, bl2, Wr2,
           Wl3, bl3, Wr3, Wlp, blp, Wrp, Wdn, bdn, Wv, bv):
    row = edge_index[0]
    col = edge_index[1]
    pad = _EPAD - _E
    ar = jnp.arange(pad, dtype=jnp.int32)
    rowp = jnp.concatenate([row, _N + ar % (_NACC - _N)])
    colp = jnp.concatenate([col, ar % _N])
    rowp = rowp.reshape(_EPAD // _C, _C)
    colp = colp.reshape(_EPAD // _C, _C)

    xa = jnp.asarray(x[:, :_H])
    xb = jnp.asarray(x[:, _H:])
    accpa, degp = _sc_agg_deg(xa, colp, rowp)
    accpb = _sc_agg(xb, colp, rowp)[0]
    h, degm = pl.pallas_call(
        _tc_layer0_body, out_shape=[_f32(_N, _H), _f32(_N, 1)])(
        accpa, accpb, degp, x, Wl0, Wr0, bl0.reshape(1, _H))

    for Wl, bl, Wr in ((Wl1, bl1, Wr1), (Wl2, bl2, Wr2), (Wl3, bl3, Wr3)):
        accp = _sc_agg(h, colp, rowp)[0]
        h = pl.pallas_call(_tc_layer_body, out_shape=_f32(_N, _H))(
            accp, degm, h, Wl, Wr, bl.reshape(1, _H))

    accp = _sc_agg(h, colp, rowp)[0]
    wrdv = jnp.concatenate([Wrp, Wdn, Wv], axis=1)
    brdv = jnp.concatenate([blp, bdn, bv]).reshape(1, 3)
    return pl.pallas_call(_tc_final_body, out_shape=_f32(_N, 3))(
        accp, degm, h, Wlp, wrdv, brdv)


# speedup vs baseline: 1.1808x; 1.1081x over previous
import jax
import jax.numpy as jnp
from jax import lax
from jax.experimental import pallas as pl
from jax.experimental.pallas import tpu as pltpu
from jax.experimental.pallas import tpu_sc as plsc

_N = 10000
_E = 320000
_DIN = 128
_H = 64

_NC = 2
_NS = 16
_NW = _NC * _NS
_C = 128
_EPAD = 327680
_PW = _EPAD // _NW
_NCH = _PW // _C
_NACC = 10240
_RPS = _NACC // _NS
_ZR = 128
_DEGW = 16


def _dotb(a, b):
    return lax.dot_general(a.astype(jnp.bfloat16), b.astype(jnp.bfloat16),
                           (((1,), (0,)), ((), ())),
                           preferred_element_type=jnp.float32)


def _make_sc_agg(nsrc, with_deg, staged=False, nbuf=4):
    mesh = plsc.VectorSubcoreMesh(core_axis_name="c", subcore_axis_name="s")
    out_types = [jax.ShapeDtypeStruct((_N, _NC * _H), jnp.float32)
                 for _ in range(nsrc)]
    scratch = [
        pltpu.VMEM((_NCH, _C), jnp.int32),
        pltpu.VMEM((_NCH, _C), jnp.int32),
        pltpu.VMEM((_ZR, _H), jnp.float32),
    ]
    for _ in range(nsrc):
        scratch += [[pltpu.VMEM((_C, _H), jnp.float32) for _ in range(nbuf)],
                    pltpu.VMEM_SHARED((_NACC, _H), jnp.float32),
                    [pltpu.SemaphoreType.DMA for _ in range(nbuf)]]
        if staged:
            scratch.append(pltpu.VMEM_SHARED((_N, _H), jnp.float32))
    if with_deg:
        out_types.append(jax.ShapeDtypeStruct((_NC, _N, _DEGW), jnp.float32))
        scratch += [
            pltpu.VMEM((_ZR, _DEGW), jnp.float32),
            pltpu.VMEM((_C, _DEGW), jnp.float32),
            pltpu.VMEM_SHARED((_NACC, _DEGW), jnp.float32),
        ]

    def body(*refs):
        y_hbm = refs[:nsrc]
        col_hbm, row_hbm = refs[nsrc:nsrc + 2]
        refs = refs[nsrc + 2:]
        acc_hbm = refs[:nsrc]
        refs = refs[nsrc:]
        if with_deg:
            deg_hbm = refs[0]
            refs = refs[1:]
        colv, rowv, zbuf = refs[:3]
        refs = refs[3:]
        stride = 4 if staged else 3
        bufs, acc_sh, sems, src_sh = [], [], [], []
        for s in range(nsrc):
            bufs.append(refs[stride * s])
            acc_sh.append(refs[stride * s + 1])
            sems.append(refs[stride * s + 2])
            if staged:
                src_sh.append(refs[stride * s + 3])
        refs = refs[stride * nsrc:]
        if with_deg:
            dzbuf, onesb, deg_sh = refs

        cid = lax.axis_index("c")
        sid = lax.axis_index("s")
        wid = cid * _NS + sid

        z16 = jnp.zeros((16,), jnp.float32)

        @pl.loop(0, _ZR)
        def _(r):
            @pl.loop(0, _H, step=16)
            def _(c0):
                zbuf[r, pl.ds(c0, 16)] = z16

        for s in range(nsrc):
            @pl.loop(0, _RPS, step=_ZR)
            def _(r0, s=s):
                pltpu.sync_copy(zbuf, acc_sh[s].at[pl.ds(sid * _RPS + r0, _ZR)])

        if with_deg:
            one16 = jnp.ones((16,), jnp.float32)

            @pl.loop(0, _ZR)
            def _(r):
                dzbuf[r, pl.ds(0, _DEGW)] = z16

            @pl.loop(0, _C)
            def _(r):
                onesb[r, pl.ds(0, _DEGW)] = one16

            @pl.loop(0, _RPS, step=_ZR)
            def _(r0):
                pltpu.sync_copy(dzbuf, deg_sh.at[pl.ds(sid * _RPS + r0, _ZR)])

        if staged:
            rps = _N // _NS
            for s in range(nsrc):
                pltpu.sync_copy(y_hbm[s].at[pl.ds(sid * rps, rps)],
                                src_sh[s].at[pl.ds(sid * rps, rps)])
            src = src_sh
        else:
            src = y_hbm

        plsc.subcore_barrier()

        pltpu.sync_copy(col_hbm.at[pl.ds(wid * _NCH, _NCH)], colv)
        pltpu.sync_copy(row_hbm.at[pl.ds(wid * _NCH, _NCH)], rowv)

        for k in range(nbuf):
            for s in range(nsrc):
                pltpu.async_copy(src[s].at[colv.at[k]], bufs[s][k],
                                 sems[s][k])

        @pl.loop(0, _NCH // nbuf)
        def _(j):
            for k in range(nbuf):
                i = nbuf * j + k
                for s in range(nsrc):
                    pltpu.make_async_copy(src[s].at[colv.at[i]], bufs[s][k],
                                          sems[s][k]).wait()
                    pltpu.sync_copy(bufs[s][k], acc_sh[s].at[rowv.at[i]],
                                    add=True)
                if with_deg:
                    pltpu.sync_copy(onesb, deg_sh.at[rowv.at[i]], add=True)

                @pl.when(i + nbuf < _NCH)
                def _(i=i, k=k):
                    for s in range(nsrc):
                        pltpu.async_copy(src[s].at[colv.at[i + nbuf]],
                                         bufs[s][k], sems[s][k])

        plsc.subcore_barrier()
        wps = _N // _NS
        for s in range(nsrc):
            pltpu.sync_copy(acc_sh[s].at[pl.ds(sid * wps, wps)],
                            acc_hbm[s].at[pl.ds(sid * wps, wps),
                                          pl.ds(cid * _H, _H)])
        if with_deg:
            pltpu.sync_copy(deg_sh.at[pl.ds(sid * wps, wps)],
                            deg_hbm.at[cid, pl.ds(sid * wps, wps)])

    return pl.kernel(body, out_type=out_types, mesh=mesh, scratch_types=scratch,
                     compiler_params=pltpu.CompilerParams(use_tc_tiling_on_sc=False))


_sc_agg_deg = _make_sc_agg(1, True)
_sc_agg = _make_sc_agg(1, False)


def _psum(accp_ref):
    return accp_ref[:, :_H] + accp_ref[:, _H:]


def _tc_layer_body(accp_ref, degm_ref, h_ref, wl_ref, wr_ref, b_ref, o_ref):
    aggn = _psum(accp_ref) / degm_ref[...]
    o_ref[...] = jnp.maximum(
        _dotb(aggn, wl_ref[...]) + b_ref[...] + _dotb(h_ref[...], wr_ref[...]),
        0.0)


def _tc_layer0_body(accpa_ref, accpb_ref, degp_ref, x_ref, wl_ref, wr_ref,
                    b_ref, o_ref, degm_ref):
    degm = jnp.maximum(degp_ref[0, :, 0:1] + degp_ref[1, :, 0:1], 1.0)
    degm_ref[...] = degm
    agg = jnp.concatenate([_psum(accpa_ref), _psum(accpb_ref)], axis=1)
    aggn = agg / degm
    o_ref[...] = jnp.maximum(
        _dotb(aggn, wl_ref[...]) + b_ref[...] + _dotb(x_ref[...], wr_ref[...]),
        0.0)


def _tc_final_body(accp_ref, degm_ref, h_ref, wlp_ref, wrdv_ref, brdv_ref,
                   o_ref):
    aggn = _psum(accp_ref) / degm_ref[...]
    h = h_ref[...]
    hd = _dotb(h, wrdv_ref[...]) + brdv_ref[...]
    probs = _dotb(aggn, wlp_ref[...]) + hd[:, 0:1]
    o_ref[...] = jnp.concatenate([probs, hd[:, 1:3]], axis=1)


def _f32(*shape):
    return jax.ShapeDtypeStruct(shape, jnp.float32)


def kernel(x, edge_index, Wl0, bl0, Wr0, Wl1, bl1, Wr1, Wl2, bl2, Wr2,
           Wl3, bl3, Wr3, Wlp, blp, Wrp, Wdn, bdn, Wv, bv):
    row = edge_index[0]
    col = edge_index[1]
    pad = _EPAD - _E
    ar = jnp.arange(pad, dtype=jnp.int32)
    rowp = jnp.concatenate([row, _N + ar % (_NACC - _N)])
    colp = jnp.concatenate([col, ar % _N])
    rowp = rowp.reshape(_EPAD // _C, _C)
    colp = colp.reshape(_EPAD // _C, _C)

    xa = jnp.asarray(x[:, :_H])
    xb = jnp.asarray(x[:, _H:])
    accpa, degp = _sc_agg_deg(xa, colp, rowp)
    accpb = _sc_agg(xb, colp, rowp)[0]
    h, degm = pl.pallas_call(
        _tc_layer0_body, out_shape=[_f32(_N, _H), _f32(_N, 1)])(
        accpa, accpb, degp, x, Wl0, Wr0, bl0.reshape(1, _H))

    for Wl, bl, Wr in ((Wl1, bl1, Wr1), (Wl2, bl2, Wr2), (Wl3, bl3, Wr3)):
        accp = _sc_agg(h, colp, rowp)[0]
        h = pl.pallas_call(_tc_layer_body, out_shape=_f32(_N, _H))(
            accp, degm, h, Wl, Wr, bl.reshape(1, _H))

    accp = _sc_agg(h, colp, rowp)[0]
    wrdv = jnp.concatenate([Wrp, Wdn, Wv], axis=1)
    brdv = jnp.concatenate([blp, bdn, bv]).reshape(1, 3)
    return pl.pallas_call(_tc_final_body, out_shape=_f32(_N, 3))(
        accp, degm, h, Wlp, wrdv, brdv)


# speedup vs baseline: 1.2097x; 1.0244x over previous
import jax
import jax.numpy as jnp
from jax import lax
from jax.experimental import pallas as pl
from jax.experimental.pallas import tpu as pltpu
from jax.experimental.pallas import tpu_sc as plsc

_N = 10000
_E = 320000
_DIN = 128
_H = 64

_NC = 2
_NS = 16
_NW = _NC * _NS
_C = 128
_EPAD = 327680
_PW = _EPAD // _NW
_NCH = _PW // _C
_NACC = 10240
_RPS = _NACC // _NS
_ZR = 128
_DEGW = 16


def _dotb(a, b):
    return lax.dot_general(a.astype(jnp.bfloat16), b.astype(jnp.bfloat16),
                           (((1,), (0,)), ((), ())),
                           preferred_element_type=jnp.float32)


def _make_sc_agg(nsrc, with_deg, staged=False, nbuf=4, idx_map=None):
    mesh = plsc.VectorSubcoreMesh(core_axis_name="c", subcore_axis_name="s")
    out_types = [jax.ShapeDtypeStruct((_N, _NC * _H), jnp.float32)
                 for _ in range(nsrc)]
    scratch = [
        pltpu.VMEM((_NCH, _C), jnp.int32),
        pltpu.VMEM((_NCH, _C), jnp.int32),
        pltpu.VMEM((_ZR, _H), jnp.float32),
    ]
    for _ in range(nsrc):
        scratch += [[pltpu.VMEM((_C, _H), jnp.float32) for _ in range(nbuf)],
                    pltpu.VMEM_SHARED((_NACC, _H), jnp.float32),
                    [pltpu.SemaphoreType.DMA for _ in range(nbuf)]]
        if staged:
            scratch.append(pltpu.VMEM_SHARED((_N, _H), jnp.float32))
    if with_deg:
        out_types.append(jax.ShapeDtypeStruct((_NC, _N, _DEGW), jnp.float32))
        scratch += [
            pltpu.VMEM((_ZR, _DEGW), jnp.float32),
            pltpu.VMEM((_C, _DEGW), jnp.float32),
            pltpu.VMEM_SHARED((_NACC, _DEGW), jnp.float32),
        ]

    def body(*refs):
        y_hbm = refs[:nsrc]
        col_hbm, row_hbm = refs[nsrc:nsrc + 2]
        refs = refs[nsrc + 2:]
        acc_hbm = refs[:nsrc]
        refs = refs[nsrc:]
        if with_deg:
            deg_hbm = refs[0]
            refs = refs[1:]
        colv, rowv, zbuf = refs[:3]
        refs = refs[3:]
        stride = 4 if staged else 3
        bufs, acc_sh, sems, src_sh = [], [], [], []
        for s in range(nsrc):
            bufs.append(refs[stride * s])
            acc_sh.append(refs[stride * s + 1])
            sems.append(refs[stride * s + 2])
            if staged:
                src_sh.append(refs[stride * s + 3])
        refs = refs[stride * nsrc:]
        if with_deg:
            dzbuf, onesb, deg_sh = refs

        cid = lax.axis_index("c")
        sid = lax.axis_index("s")
        wid = cid * _NS + sid

        z16 = jnp.zeros((16,), jnp.float32)

        @pl.loop(0, _ZR)
        def _(r):
            @pl.loop(0, _H, step=16)
            def _(c0):
                zbuf[r, pl.ds(c0, 16)] = z16

        for s in range(nsrc):
            @pl.loop(0, _RPS, step=_ZR)
            def _(r0, s=s):
                pltpu.sync_copy(zbuf, acc_sh[s].at[pl.ds(sid * _RPS + r0, _ZR)])

        if with_deg:
            one16 = jnp.ones((16,), jnp.float32)

            @pl.loop(0, _ZR)
            def _(r):
                dzbuf[r, pl.ds(0, _DEGW)] = z16

            @pl.loop(0, _C)
            def _(r):
                onesb[r, pl.ds(0, _DEGW)] = one16

            @pl.loop(0, _RPS, step=_ZR)
            def _(r0):
                pltpu.sync_copy(dzbuf, deg_sh.at[pl.ds(sid * _RPS + r0, _ZR)])

        if staged:
            rps = _N // _NS
            for s in range(nsrc):
                pltpu.sync_copy(y_hbm[s].at[pl.ds(sid * rps, rps)],
                                src_sh[s].at[pl.ds(sid * rps, rps)])
            src = src_sh
        else:
            src = y_hbm

        plsc.subcore_barrier()

        pltpu.sync_copy(col_hbm.at[pl.ds(wid * _NCH, _NCH)], colv)
        pltpu.sync_copy(row_hbm.at[pl.ds(wid * _NCH, _NCH)], rowv)

        if idx_map is not None:
            scale, off = idx_map

            @pl.loop(0, _NCH)
            def _(r):
                @pl.loop(0, _C, step=16)
                def _(c0):
                    colv[r, pl.ds(c0, 16)] = (
                        colv[r, pl.ds(c0, 16)] * scale + off)

        for k in range(nbuf):
            for s in range(nsrc):
                pltpu.async_copy(src[s].at[colv.at[k]], bufs[s][k],
                                 sems[s][k])

        @pl.loop(0, _NCH // nbuf)
        def _(j):
            for k in range(nbuf):
                i = nbuf * j + k
                for s in range(nsrc):
                    pltpu.make_async_copy(src[s].at[colv.at[i]], bufs[s][k],
                                          sems[s][k]).wait()
                    pltpu.sync_copy(bufs[s][k], acc_sh[s].at[rowv.at[i]],
                                    add=True)
                if with_deg:
                    pltpu.sync_copy(onesb, deg_sh.at[rowv.at[i]], add=True)

                @pl.when(i + nbuf < _NCH)
                def _(i=i, k=k):
                    for s in range(nsrc):
                        pltpu.async_copy(src[s].at[colv.at[i + nbuf]],
                                         bufs[s][k], sems[s][k])

        plsc.subcore_barrier()
        wps = _N // _NS
        for s in range(nsrc):
            pltpu.sync_copy(acc_sh[s].at[pl.ds(sid * wps, wps)],
                            acc_hbm[s].at[pl.ds(sid * wps, wps),
                                          pl.ds(cid * _H, _H)])
        if with_deg:
            pltpu.sync_copy(deg_sh.at[pl.ds(sid * wps, wps)],
                            deg_hbm.at[cid, pl.ds(sid * wps, wps)])

    return pl.kernel(body, out_type=out_types, mesh=mesh, scratch_types=scratch,
                     compiler_params=pltpu.CompilerParams(use_tc_tiling_on_sc=False))


_sc_agg_deg = _make_sc_agg(1, True, idx_map=(2, 0))
_sc_agg_odd = _make_sc_agg(1, False, idx_map=(2, 1))
_sc_agg = _make_sc_agg(1, False)


def _psum(accp_ref):
    return accp_ref[:, :_H] + accp_ref[:, _H:]


def _tc_layer_body(accp_ref, degm_ref, h_ref, wl_ref, wr_ref, b_ref, o_ref):
    aggn = _psum(accp_ref) / degm_ref[...]
    o_ref[...] = jnp.maximum(
        _dotb(aggn, wl_ref[...]) + b_ref[...] + _dotb(h_ref[...], wr_ref[...]),
        0.0)


def _tc_layer0_body(accpa_ref, accpb_ref, degp_ref, x_ref, wl_ref, wr_ref,
                    b_ref, o_ref, degm_ref):
    degm = jnp.maximum(degp_ref[0, :, 0:1] + degp_ref[1, :, 0:1], 1.0)
    degm_ref[...] = degm
    agg = jnp.concatenate([_psum(accpa_ref), _psum(accpb_ref)], axis=1)
    aggn = agg / degm
    o_ref[...] = jnp.maximum(
        _dotb(aggn, wl_ref[...]) + b_ref[...] + _dotb(x_ref[...], wr_ref[...]),
        0.0)


def _tc_final_body(accp_ref, degm_ref, h_ref, wlp_ref, wrdv_ref, brdv_ref,
                   o_ref):
    aggn = _psum(accp_ref) / degm_ref[...]
    h = h_ref[...]
    hd = _dotb(h, wrdv_ref[...]) + brdv_ref[...]
    probs = _dotb(aggn, wlp_ref[...]) + hd[:, 0:1]
    o_ref[...] = jnp.concatenate([probs, hd[:, 1:3]], axis=1)


def _f32(*shape):
    return jax.ShapeDtypeStruct(shape, jnp.float32)


def kernel(x, edge_index, Wl0, bl0, Wr0, Wl1, bl1, Wr1, Wl2, bl2, Wr2,
           Wl3, bl3, Wr3, Wlp, blp, Wrp, Wdn, bdn, Wv, bv):
    row = edge_index[0]
    col = edge_index[1]
    pad = _EPAD - _E
    ar = jnp.arange(pad, dtype=jnp.int32)
    rowp = jnp.concatenate([row, _N + ar % (_NACC - _N)])
    colp = jnp.concatenate([col, ar % _N])
    rowp = rowp.reshape(_EPAD // _C, _C)
    colp = colp.reshape(_EPAD // _C, _C)

    x64 = x.reshape(2 * _N, _H)
    accpa, degp = _sc_agg_deg(x64, colp, rowp)
    accpb = _sc_agg_odd(x64, colp, rowp)[0]
    h, degm = pl.pallas_call(
        _tc_layer0_body, out_shape=[_f32(_N, _H), _f32(_N, 1)])(
        accpa, accpb, degp, x, Wl0, Wr0, bl0.reshape(1, _H))

    for Wl, bl, Wr in ((Wl1, bl1, Wr1), (Wl2, bl2, Wr2), (Wl3, bl3, Wr3)):
        accp = _sc_agg(h, colp, rowp)[0]
        h = pl.pallas_call(_tc_layer_body, out_shape=_f32(_N, _H))(
            accp, degm, h, Wl, Wr, bl.reshape(1, _H))

    accp = _sc_agg(h, colp, rowp)[0]
    wrdv = jnp.concatenate([Wrp, Wdn, Wv], axis=1)
    brdv = jnp.concatenate([blp, bdn, bv]).reshape(1, 3)
    return pl.pallas_call(_tc_final_body, out_shape=_f32(_N, 3))(
        accp, degm, h, Wlp, wrdv, brdv)


# speedup vs baseline: 1.2117x; 1.0017x over previous
import jax
import jax.numpy as jnp
from jax import lax
from jax.experimental import pallas as pl
from jax.experimental.pallas import tpu as pltpu
from jax.experimental.pallas import tpu_sc as plsc

_N = 10000
_E = 320000
_DIN = 128
_H = 64

_NC = 2
_NS = 16
_NW = _NC * _NS
_C = 128
_EPAD = 327680
_PW = _EPAD // _NW
_NCH = _PW // _C
_NACC = 10240
_RPS = _NACC // _NS
_ZR = 128
_DEGW = 16


def _dotb(a, b):
    return lax.dot_general(a.astype(jnp.bfloat16), b.astype(jnp.bfloat16),
                           (((1,), (0,)), ((), ())),
                           preferred_element_type=jnp.float32)


def _make_sc_agg(nsrc, with_deg, staged=False, nbuf=4, idx_map=None):
    mesh = plsc.VectorSubcoreMesh(core_axis_name="c", subcore_axis_name="s")
    out_types = [jax.ShapeDtypeStruct((_N, _NC * _H), jnp.float32)
                 for _ in range(nsrc)]
    scratch = [
        pltpu.VMEM((_NCH, _C), jnp.int32),
        pltpu.VMEM((_NCH, _C), jnp.int32),
        pltpu.VMEM((_ZR, _H), jnp.float32),
    ]
    for _ in range(nsrc):
        scratch += [[pltpu.VMEM((_C, _H), jnp.float32) for _ in range(nbuf)],
                    pltpu.VMEM_SHARED((_NACC, _H), jnp.float32),
                    [pltpu.SemaphoreType.DMA for _ in range(nbuf)]]
        if staged:
            scratch.append(pltpu.VMEM_SHARED((_N, _H), jnp.float32))
    if with_deg:
        out_types.append(jax.ShapeDtypeStruct((_NC, _N, _DEGW), jnp.float32))
        scratch += [
            pltpu.VMEM((_ZR, _DEGW), jnp.float32),
            pltpu.VMEM((_C, _DEGW), jnp.float32),
            pltpu.VMEM_SHARED((_NACC, _DEGW), jnp.float32),
        ]

    def body(*refs):
        y_hbm = refs[:nsrc]
        col_hbm, row_hbm = refs[nsrc:nsrc + 2]
        refs = refs[nsrc + 2:]
        acc_hbm = refs[:nsrc]
        refs = refs[nsrc:]
        if with_deg:
            deg_hbm = refs[0]
            refs = refs[1:]
        colv, rowv, zbuf = refs[:3]
        refs = refs[3:]
        stride = 4 if staged else 3
        bufs, acc_sh, sems, src_sh = [], [], [], []
        for s in range(nsrc):
            bufs.append(refs[stride * s])
            acc_sh.append(refs[stride * s + 1])
            sems.append(refs[stride * s + 2])
            if staged:
                src_sh.append(refs[stride * s + 3])
        refs = refs[stride * nsrc:]
        if with_deg:
            dzbuf, onesb, deg_sh = refs

        cid = lax.axis_index("c")
        sid = lax.axis_index("s")
        wid = cid * _NS + sid

        z16 = jnp.zeros((16,), jnp.float32)

        @pl.loop(0, _ZR)
        def _(r):
            @pl.loop(0, _H, step=16)
            def _(c0):
                zbuf[r, pl.ds(c0, 16)] = z16

        for s in range(nsrc):
            @pl.loop(0, _RPS, step=_ZR)
            def _(r0, s=s):
                pltpu.sync_copy(zbuf, acc_sh[s].at[pl.ds(sid * _RPS + r0, _ZR)])

        if with_deg:
            one16 = jnp.ones((16,), jnp.float32)

            @pl.loop(0, _ZR)
            def _(r):
                dzbuf[r, pl.ds(0, _DEGW)] = z16

            @pl.loop(0, _C)
            def _(r):
                onesb[r, pl.ds(0, _DEGW)] = one16

            @pl.loop(0, _RPS, step=_ZR)
            def _(r0):
                pltpu.sync_copy(dzbuf, deg_sh.at[pl.ds(sid * _RPS + r0, _ZR)])

        if staged:
            rps = _N // _NS
            for s in range(nsrc):
                pltpu.sync_copy(y_hbm[s].at[pl.ds(sid * rps, rps)],
                                src_sh[s].at[pl.ds(sid * rps, rps)])
            src = src_sh
        else:
            src = y_hbm

        pltpu.sync_copy(col_hbm.at[pl.ds(wid * _NCH, _NCH)], colv)
        pltpu.sync_copy(row_hbm.at[pl.ds(wid * _NCH, _NCH)], rowv)

        if idx_map is not None:
            scale, off = idx_map

            @pl.loop(0, _NCH)
            def _(r):
                @pl.loop(0, _C, step=16)
                def _(c0):
                    colv[r, pl.ds(c0, 16)] = (
                        colv[r, pl.ds(c0, 16)] * scale + off)

        for k in range(nbuf):
            for s in range(nsrc):
                pltpu.async_copy(src[s].at[colv.at[k]], bufs[s][k],
                                 sems[s][k])

        plsc.subcore_barrier()

        @pl.loop(0, _NCH // nbuf)
        def _(j):
            for k in range(nbuf):
                i = nbuf * j + k
                for s in range(nsrc):
                    pltpu.make_async_copy(src[s].at[colv.at[i]], bufs[s][k],
                                          sems[s][k]).wait()
                    pltpu.sync_copy(bufs[s][k], acc_sh[s].at[rowv.at[i]],
                                    add=True)
                if with_deg:
                    pltpu.sync_copy(onesb, deg_sh.at[rowv.at[i]], add=True)

                @pl.when(i + nbuf < _NCH)
                def _(i=i, k=k):
                    for s in range(nsrc):
                        pltpu.async_copy(src[s].at[colv.at[i + nbuf]],
                                         bufs[s][k], sems[s][k])

        plsc.subcore_barrier()
        wps = _N // _NS
        for s in range(nsrc):
            pltpu.sync_copy(acc_sh[s].at[pl.ds(sid * wps, wps)],
                            acc_hbm[s].at[pl.ds(sid * wps, wps),
                                          pl.ds(cid * _H, _H)])
        if with_deg:
            pltpu.sync_copy(deg_sh.at[pl.ds(sid * wps, wps)],
                            deg_hbm.at[cid, pl.ds(sid * wps, wps)])

    return pl.kernel(body, out_type=out_types, mesh=mesh, scratch_types=scratch,
                     compiler_params=pltpu.CompilerParams(use_tc_tiling_on_sc=False))


_sc_agg_deg = _make_sc_agg(1, True, idx_map=(2, 0))
_sc_agg_odd = _make_sc_agg(1, False, idx_map=(2, 1))
_sc_agg = _make_sc_agg(1, False)


def _psum(accp_ref):
    return accp_ref[:, :_H] + accp_ref[:, _H:]


def _tc_layer_body(accp_ref, degm_ref, h_ref, wl_ref, wr_ref, b_ref, o_ref):
    aggn = _psum(accp_ref) / degm_ref[...]
    o_ref[...] = jnp.maximum(
        _dotb(aggn, wl_ref[...]) + b_ref[...] + _dotb(h_ref[...], wr_ref[...]),
        0.0)


def _tc_layer0_body(accpa_ref, accpb_ref, degp_ref, x_ref, wl_ref, wr_ref,
                    b_ref, o_ref, degm_ref):
    degm = jnp.maximum(degp_ref[0, :, 0:1] + degp_ref[1, :, 0:1], 1.0)
    degm_ref[...] = degm
    agg = jnp.concatenate([_psum(accpa_ref), _psum(accpb_ref)], axis=1)
    aggn = agg / degm
    o_ref[...] = jnp.maximum(
        _dotb(aggn, wl_ref[...]) + b_ref[...] + _dotb(x_ref[...], wr_ref[...]),
        0.0)


def _tc_final_body(accp_ref, degm_ref, h_ref, wlp_ref, wrdv_ref, brdv_ref,
                   o_ref):
    aggn = _psum(accp_ref) / degm_ref[...]
    h = h_ref[...]
    hd = _dotb(h, wrdv_ref[...]) + brdv_ref[...]
    probs = _dotb(aggn, wlp_ref[...]) + hd[:, 0:1]
    o_ref[...] = jnp.concatenate([probs, hd[:, 1:3]], axis=1)


def _f32(*shape):
    return jax.ShapeDtypeStruct(shape, jnp.float32)


def kernel(x, edge_index, Wl0, bl0, Wr0, Wl1, bl1, Wr1, Wl2, bl2, Wr2,
           Wl3, bl3, Wr3, Wlp, blp, Wrp, Wdn, bdn, Wv, bv):
    row = edge_index[0]
    col = edge_index[1]
    pad = _EPAD - _E
    ar = jnp.arange(pad, dtype=jnp.int32)
    rowp = jnp.concatenate([row, _N + ar % (_NACC - _N)])
    colp = jnp.concatenate([col, ar % _N])
    rowp = rowp.reshape(_EPAD // _C, _C)
    colp = colp.reshape(_EPAD // _C, _C)

    x64 = x.reshape(2 * _N, _H)
    accpa, degp = _sc_agg_deg(x64, colp, rowp)
    accpb = _sc_agg_odd(x64, colp, rowp)[0]
    h, degm = pl.pallas_call(
        _tc_layer0_body, out_shape=[_f32(_N, _H), _f32(_N, 1)])(
        accpa, accpb, degp, x, Wl0, Wr0, bl0.reshape(1, _H))

    for Wl, bl, Wr in ((Wl1, bl1, Wr1), (Wl2, bl2, Wr2), (Wl3, bl3, Wr3)):
        accp = _sc_agg(h, colp, rowp)[0]
        h = pl.pallas_call(_tc_layer_body, out_shape=_f32(_N, _H))(
            accp, degm, h, Wl, Wr, bl.reshape(1, _H))

    accp = _sc_agg(h, colp, rowp)[0]
    wrdv = jnp.concatenate([Wrp, Wdn, Wv], axis=1)
    brdv = jnp.concatenate([blp, bdn, bv]).reshape(1, 3)
    return pl.pallas_call(_tc_final_body, out_shape=_f32(_N, 3))(
        accp, degm, h, Wlp, wrdv, brdv)
